# baseline (speedup 1.0000x reference)
import functools

import jax
import jax.numpy as jnp
from jax import lax
from jax.experimental import pallas as pl
from jax.experimental.pallas import tpu as pltpu
from jax.experimental.pallas import tpu_sc as plsc

NUM_CORES = 2
NUM_SUBCORES = 16
LANES = 16
NUM_WORKERS = NUM_CORES * NUM_SUBCORES

BATCH = 4096
N_A = 3072
N_B = 2048
N_OUT = 3584

ROWS_PER_WORKER = BATCH // NUM_WORKERS
CHUNK_R = 8
NUM_CHUNKS = ROWS_PER_WORKER // CHUNK_R
N_J = N_OUT // LANES


def _sc_gather(a_flat, b_flat, ia, ib, wa, wb):
    mesh = plsc.VectorSubcoreMesh(
        core_axis_name="c", subcore_axis_name="s",
        num_cores=NUM_CORES, num_subcores=NUM_SUBCORES)

    @functools.partial(
        pl.kernel,
        out_type=jax.ShapeDtypeStruct((BATCH * N_OUT,), jnp.float32),
        mesh=mesh,
        compiler_params=pltpu.CompilerParams(needs_layout_passes=False),
        scratch_types=[
            pltpu.VMEM((N_OUT,), jnp.int32),
            pltpu.VMEM((N_OUT,), jnp.int32),
            pltpu.VMEM((N_OUT,), jnp.float32),
            pltpu.VMEM((N_OUT,), jnp.float32),
            pltpu.VMEM((CHUNK_R * N_A,), jnp.float32),
            pltpu.VMEM((CHUNK_R * N_B,), jnp.float32),
            pltpu.VMEM((CHUNK_R * N_OUT,), jnp.float32),
        ],
    )
    def k(a_hbm, b_hbm, ia_hbm, ib_hbm, wa_hbm, wb_hbm, out_hbm,
          ia_v, ib_v, wa_v, wb_v, a_v, b_v, o_v):
        wid = lax.axis_index("s") * NUM_CORES + lax.axis_index("c")
        row0 = wid * ROWS_PER_WORKER
        pltpu.sync_copy(ia_hbm, ia_v)
        pltpu.sync_copy(ib_hbm, ib_v)
        pltpu.sync_copy(wa_hbm, wa_v)
        pltpu.sync_copy(wb_hbm, wb_v)

        def chunk_body(kc, _):
            r0 = row0 + kc * CHUNK_R
            pltpu.sync_copy(a_hbm.at[pl.ds(r0 * N_A, CHUNK_R * N_A)], a_v)
            pltpu.sync_copy(b_hbm.at[pl.ds(r0 * N_B, CHUNK_R * N_B)], b_v)

            def j_body(j, _):
                c0 = j * LANES
                iav = ia_v[pl.ds(c0, LANES)]
                ibv = ib_v[pl.ds(c0, LANES)]
                wav = wa_v[pl.ds(c0, LANES)]
                wbv = wb_v[pl.ds(c0, LANES)]
                for r in range(CHUNK_R):
                    va = plsc.load_gather(a_v, [iav + (r * N_A)])
                    vb = plsc.load_gather(b_v, [ibv + (r * N_B)])
                    o_v[pl.ds(r * N_OUT + c0, LANES)] = va * wav + vb * wbv
                return 0

            lax.fori_loop(0, N_J, j_body, 0)
            pltpu.sync_copy(o_v, out_hbm.at[pl.ds(r0 * N_OUT, CHUNK_R * N_OUT)])
            return 0

        lax.fori_loop(0, NUM_CHUNKS, chunk_body, 0)

    return k(a_flat, b_flat, ia, ib, wa, wb)


def kernel(a, b, channels_a, channels_b, channels_out):
    ch_a = channels_a.astype(jnp.int32)
    ch_b = channels_b.astype(jnp.int32)
    ch_out = channels_out.astype(jnp.int32)

    pa = jnp.minimum(jnp.searchsorted(ch_a, ch_out), N_A - 1).astype(jnp.int32)
    in_a = ch_a[pa] == ch_out
    pb = jnp.minimum(jnp.searchsorted(ch_b, ch_out), N_B - 1).astype(jnp.int32)
    in_b = ch_b[pb] == ch_out

    ia = jnp.where(in_a, pa, 0)
    ib = jnp.where(in_b, pb, 0)
    wa = in_a.astype(jnp.float32)
    wb = in_b.astype(jnp.float32)
    out = _sc_gather(a.reshape(-1), b.reshape(-1), ia, ib, wa, wb)
    return out.reshape(BATCH, N_OUT)


# speedup vs baseline: 2.2050x; 2.2050x over previous
import functools

import jax
import jax.numpy as jnp
from jax import lax
from jax.experimental import pallas as pl
from jax.experimental.pallas import tpu as pltpu
from jax.experimental.pallas import tpu_sc as plsc

NUM_CORES = 2
NUM_SUBCORES = 16
LANES = 16
NUM_WORKERS = NUM_CORES * NUM_SUBCORES

BATCH = 4096
N_A = 3072
N_B = 2048
N_OUT = 3584

ROWS_PER_WORKER = BATCH // NUM_WORKERS
CHUNK_R = 4
NUM_CHUNKS = ROWS_PER_WORKER // CHUNK_R
N_J = N_OUT // LANES


def _sc_gather(a, b, ia, ib, wa, wb):
    mesh = plsc.VectorSubcoreMesh(
        core_axis_name="c", subcore_axis_name="s",
        num_cores=NUM_CORES, num_subcores=NUM_SUBCORES)

    @functools.partial(
        pl.kernel,
        out_type=jax.ShapeDtypeStruct((BATCH, N_OUT), jnp.float32),
        mesh=mesh,
        compiler_params=pltpu.CompilerParams(needs_layout_passes=False),
        scratch_types=[
            pltpu.VMEM((N_OUT,), jnp.int32),
            pltpu.VMEM((N_OUT,), jnp.int32),
            pltpu.VMEM((N_OUT,), jnp.float32),
            pltpu.VMEM((N_OUT,), jnp.float32),
            pltpu.VMEM((CHUNK_R, N_A), jnp.float32),
            pltpu.VMEM((CHUNK_R, N_A), jnp.float32),
            pltpu.VMEM((CHUNK_R, N_B), jnp.float32),
            pltpu.VMEM((CHUNK_R, N_B), jnp.float32),
            pltpu.VMEM((CHUNK_R, N_OUT), jnp.float32),
            pltpu.VMEM((CHUNK_R, N_OUT), jnp.float32),
            pltpu.SemaphoreType.DMA,
            pltpu.SemaphoreType.DMA,
            pltpu.SemaphoreType.DMA,
            pltpu.SemaphoreType.DMA,
            pltpu.SemaphoreType.DMA,
            pltpu.SemaphoreType.DMA,
        ],
    )
    def k(a_hbm, b_hbm, ia_hbm, ib_hbm, wa_hbm, wb_hbm, out_hbm,
          ia_v, ib_v, wa_v, wb_v,
          a_v0, a_v1, b_v0, b_v1, o_v0, o_v1,
          sa0, sa1, sb0, sb1, so0, so1):
        a_bufs, b_bufs, o_bufs = (a_v0, a_v1), (b_v0, b_v1), (o_v0, o_v1)
        sa, sb, so = (sa0, sa1), (sb0, sb1), (so0, so1)

        wid = lax.axis_index("s") * NUM_CORES + lax.axis_index("c")
        row0 = wid * ROWS_PER_WORKER
        pltpu.sync_copy(ia_hbm, ia_v)
        pltpu.sync_copy(ib_hbm, ib_v)
        pltpu.sync_copy(wa_hbm, wa_v)
        pltpu.sync_copy(wb_hbm, wb_v)

        def start_in(kc, p):
            r0 = row0 + kc * CHUNK_R
            pltpu.async_copy(a_hbm.at[pl.ds(r0, CHUNK_R), :], a_bufs[p], sa[p])
            pltpu.async_copy(b_hbm.at[pl.ds(r0, CHUNK_R), :], b_bufs[p], sb[p])

        def wait_in(p):
            pltpu.make_async_copy(
                a_hbm.at[pl.ds(0, CHUNK_R), :], a_bufs[p], sa[p]).wait()
            pltpu.make_async_copy(
                b_hbm.at[pl.ds(0, CHUNK_R), :], b_bufs[p], sb[p]).wait()

        def wait_out(p):
            pltpu.make_async_copy(
                o_bufs[p], out_hbm.at[pl.ds(0, CHUNK_R), :], so[p]).wait()

        start_in(0, 0)
        start_in(1, 1)

        def pair_body(kp, _):
            for p in range(2):
                kc = kp * 2 + p
                wait_in(p)

                @pl.when(kp > 0)
                def _():
                    wait_out(p)

                o_v = o_bufs[p]
                a_v = a_bufs[p]
                b_v = b_bufs[p]

                def j_body(j, _):
                    c0 = j * LANES
                    iav = ia_v[pl.ds(c0, LANES)]
                    ibv = ib_v[pl.ds(c0, LANES)]
                    wav = wa_v[pl.ds(c0, LANES)]
                    wbv = wb_v[pl.ds(c0, LANES)]
                    for r in range(CHUNK_R):
                        rvec = jnp.full((LANES,), r, jnp.int32)
                        va = plsc.load_gather(a_v, [rvec, iav])
                        vb = plsc.load_gather(b_v, [rvec, ibv])
                        o_v[r, pl.ds(c0, LANES)] = va * wav + vb * wbv
                    return 0

                lax.fori_loop(0, N_J, j_body, 0, unroll=2)

                kn = jnp.where(kc + 2 < NUM_CHUNKS, kc + 2, kc)
                start_in(kn, p)

                r0 = row0 + kc * CHUNK_R
                pltpu.async_copy(o_v, out_hbm.at[pl.ds(r0, CHUNK_R), :], so[p])
            return 0

        lax.fori_loop(0, NUM_CHUNKS // 2, pair_body, 0)

        for p in range(2):
            wait_in(p)
            wait_out(p)

    return k(a, b, ia, ib, wa, wb)


def kernel(a, b, channels_a, channels_b, channels_out):
    ch_a = channels_a.astype(jnp.int32)
    ch_b = channels_b.astype(jnp.int32)
    ch_out = channels_out.astype(jnp.int32)

    co = ch_out[:, None]
    pa = jnp.sum((ch_a[None, :] < co).astype(jnp.int32), axis=1)
    in_a = jnp.any(ch_a[None, :] == co, axis=1)
    pb = jnp.sum((ch_b[None, :] < co).astype(jnp.int32), axis=1)
    in_b = jnp.any(ch_b[None, :] == co, axis=1)

    ia = jnp.where(in_a, pa, 0).astype(jnp.int32)
    ib = jnp.where(in_b, pb, 0).astype(jnp.int32)
    wa = in_a.astype(jnp.float32)
    wb = in_b.astype(jnp.float32)
    return _sc_gather(a, b, ia, ib, wa, wb)
